# baseline (speedup 1.0000x reference)
import functools

import jax
import jax.numpy as jnp
from jax import lax
from jax.experimental import pallas as pl
from jax.experimental.pallas import tpu as pltpu
from jax.experimental.pallas import tpu_sc as plsc

CUT = 7.0


def _poscart_body(pos_ref, cell_ref, out_ref):
    p = pos_ref[...]
    c = cell_ref[...]
    out_ref[...] = (p[:, 0:1] * c[:, 0, :] + p[:, 1:2] * c[:, 1, :]
                    + p[:, 2:3] * c[:, 2, :])


def _tc_poscart(pos, cell_rep, tn=400):
    n = pos.shape[0]
    return pl.pallas_call(
        _poscart_body,
        grid=(n // tn,),
        in_specs=[pl.BlockSpec((tn, 3), lambda i: (i, 0)),
                  pl.BlockSpec((tn, 3, 3), lambda i: (i, 0, 0))],
        out_specs=pl.BlockSpec((tn, 3), lambda i: (i, 0)),
        out_shape=jax.ShapeDtypeStruct((n, 3), jnp.float32),
    )(pos, cell_rep)


def _embed_body(an_ref, emb_ref, out_ref):
    an = an_ref[...]
    oh = (an == lax.broadcasted_iota(jnp.int32, (1, 128), 1)).astype(jnp.float32)
    out_ref[...] = jnp.dot(oh, emb_ref[...], preferred_element_type=jnp.float32, precision=jax.lax.Precision.HIGHEST)


def _tc_embed(atomic_numbers, emb_pad, tn=400):
    n = atomic_numbers.shape[0]
    d = emb_pad.shape[1]
    an2 = atomic_numbers.astype(jnp.int32).reshape(n, 1)
    return pl.pallas_call(
        _embed_body,
        grid=(n // tn,),
        in_specs=[pl.BlockSpec((tn, 1), lambda i: (i, 0)),
                  pl.BlockSpec((128, d), lambda i: (0, 0))],
        out_specs=pl.BlockSpec((tn, d), lambda i: (i, 0)),
        out_shape=jax.ShapeDtypeStruct((n, d), jnp.float32),
    )(an2, emb_pad)


def _msgagg_body(nr, te, w_ref, s2_ref, hsrc_ref, dl_ref, wrbf_ref, wmsg_ref,
                 out_ref):
    i = pl.program_id(0)
    s2 = s2_ref[...]
    dist = jnp.sqrt(s2 + 1e-12)
    centers = (CUT / (nr - 1)) * lax.broadcasted_iota(
        jnp.int32, (1, nr), 1).astype(jnp.float32)
    gamma = (nr / CUT) ** 2
    rbf = jnp.exp(-gamma * (dist - centers) ** 2)
    m = jax.nn.silu(jnp.dot(rbf, wrbf_ref[...],
                            preferred_element_type=jnp.float32))
    x = hsrc_ref[...] * m
    msg = jax.nn.silu(jnp.dot(x, wmsg_ref[...],
                              preferred_element_type=jnp.float32))
    oh = (dl_ref[...] == lax.broadcasted_iota(jnp.int32, (1, te), 1)
          ).astype(jnp.float32)
    partial = lax.dot_general(oh, msg, (((0,), (0,)), ((), ())),
                              preferred_element_type=jnp.float32)

    @pl.when((i == 0) | (w_ref[i] != w_ref[jnp.maximum(i - 1, 0)]))
    def _init():
        out_ref[...] = jnp.zeros_like(out_ref)

    out_ref[...] += partial


def _tc_msg_agg(w_map, s2col, hsrc, dstloc, w_rbf, w_msg, npad, te=256):
    ep, d = hsrc.shape
    nr = w_rbf.shape[0]
    grid = ep // te
    return pl.pallas_call(
        functools.partial(_msgagg_body, nr, te),
        grid_spec=pltpu.PrefetchScalarGridSpec(
            num_scalar_prefetch=1,
            grid=(grid,),
            in_specs=[pl.BlockSpec((te, 1), lambda i, w: (i, 0)),
                      pl.BlockSpec((te, d), lambda i, w: (i, 0)),
                      pl.BlockSpec((te, 1), lambda i, w: (i, 0)),
                      pl.BlockSpec((nr, d), lambda i, w: (0, 0)),
                      pl.BlockSpec((d, d), lambda i, w: (0, 0))],
            out_specs=pl.BlockSpec((te, d), lambda i, w: (w[i], 0)),
        ),
        out_shape=jax.ShapeDtypeStruct((npad, d), jnp.float32),
    )(w_map, s2col, hsrc, dstloc, w_rbf, w_msg)


def _upd_body(h_ref, agg_ref, w_ref, out_ref):
    x = h_ref[...] + agg_ref[...]
    out_ref[...] = jax.nn.silu(jnp.dot(x, w_ref[...],
                                       preferred_element_type=jnp.float32, precision=jax.lax.Precision.HIGHEST))


def _tc_update(h, agg_pad, w_upd, tn=400):
    n, d = h.shape
    return pl.pallas_call(
        _upd_body,
        grid=(n // tn,),
        in_specs=[pl.BlockSpec((tn, d), lambda i: (i, 0)),
                  pl.BlockSpec((tn, d), lambda i: (i, 0)),
                  pl.BlockSpec((d, d), lambda i: (0, 0))],
        out_specs=pl.BlockSpec((tn, d), lambda i: (i, 0)),
        out_shape=jax.ShapeDtypeStruct((n, d), jnp.float32),
    )(h, agg_pad, w_upd)


def _pool_body(apg, h_ref, out_ref):
    h = h_ref[...]
    tn = h.shape[0]
    gi = lax.broadcasted_iota(jnp.int32, (tn // apg, tn), 0)
    ii = lax.broadcasted_iota(jnp.int32, (tn // apg, tn), 1)
    sel = ((ii // apg) == gi).astype(jnp.float32)
    out_ref[...] = jnp.dot(sel, h, preferred_element_type=jnp.float32, precision=jax.lax.Precision.HIGHEST) * (1.0 / apg)


def _tc_pool(h, b, tn=400):
    n, d = h.shape
    apg = n // b
    gpb = tn // apg
    return pl.pallas_call(
        functools.partial(_pool_body, apg),
        grid=(n // tn,),
        in_specs=[pl.BlockSpec((tn, d), lambda i: (i, 0))],
        out_specs=pl.BlockSpec((gpb, d), lambda i: (i, 0)),
        out_shape=jax.ShapeDtypeStruct((b, d), jnp.float32),
    )(h)


def _head_body(p_ref, w1_ref, b1_ref, w2_ref, b2_ref, wd_ref, bd_ref,
               pred_ref, repr_ref):
    p = p_ref[...]
    hdn = jax.nn.silu(jnp.dot(p, w1_ref[...],
                              preferred_element_type=jnp.float32, precision=jax.lax.Precision.HIGHEST) + b1_ref[...])
    pred_ref[...] = jnp.dot(hdn, w2_ref[...],
                            preferred_element_type=jnp.float32, precision=jax.lax.Precision.HIGHEST) + b2_ref[...]
    repr_ref[...] = jnp.dot(p, wd_ref[...],
                            preferred_element_type=jnp.float32, precision=jax.lax.Precision.HIGHEST) + bd_ref[...]


def _tc_head(pooled, w1, b1, w2, b2, wd, bd):
    b, d = pooled.shape
    dh = w1.shape[1]
    td = wd.shape[1]
    return pl.pallas_call(
        _head_body,
        grid=(1,),
        in_specs=[pl.BlockSpec((b, d), lambda i: (0, 0)),
                  pl.BlockSpec((d, dh), lambda i: (0, 0)),
                  pl.BlockSpec((1, dh), lambda i: (0, 0)),
                  pl.BlockSpec((dh, 1), lambda i: (0, 0)),
                  pl.BlockSpec((1, 1), lambda i: (0, 0)),
                  pl.BlockSpec((d, td), lambda i: (0, 0)),
                  pl.BlockSpec((1, td), lambda i: (0, 0))],
        out_specs=[pl.BlockSpec((b, 1), lambda i: (0, 0)),
                   pl.BlockSpec((b, td), lambda i: (0, 0))],
        out_shape=[jax.ShapeDtypeStruct((b, 1), jnp.float32),
                   jax.ShapeDtypeStruct((b, td), jnp.float32)],
    )(pooled, w1, b1.reshape(1, dh), w2, b2.reshape(1, 1), wd,
      bd.reshape(1, td))


_SC_MESH = dict(core_axis_name="c", subcore_axis_name="s")


def _sc_dist(pos_flat, n, src_p, dst_p):
    ep = src_p.shape[0]
    per_tile = ep // 32
    ch = 64
    n_chunks = per_tile // ch

    @functools.partial(
        pl.kernel,
        mesh=plsc.VectorSubcoreMesh(**_SC_MESH),
        compiler_params=pltpu.CompilerParams(needs_layout_passes=False),
        out_type=jax.ShapeDtypeStruct((ep,), jnp.float32),
        scratch_types=[pltpu.VMEM((n * 3,), jnp.float32),
                       pltpu.VMEM((ch,), jnp.int32),
                       pltpu.VMEM((ch,), jnp.int32),
                       pltpu.VMEM((ch,), jnp.float32)],
    )
    def k(pos_hbm, src_hbm, dst_hbm, out_hbm, tab_v, si_v, di_v, o_v):
        wid = lax.axis_index("s") * 2 + lax.axis_index("c")
        base = wid * per_tile
        pltpu.sync_copy(pos_hbm, tab_v)

        def chunk(j, carry):
            e0 = base + j * ch
            pltpu.sync_copy(src_hbm.at[pl.ds(e0, ch)], si_v)
            pltpu.sync_copy(dst_hbm.at[pl.ds(e0, ch)], di_v)
            nmax = jnp.full((16,), n - 1, jnp.int32)
            for v in range(ch // 16):
                draw = di_v[pl.ds(v * 16, 16)]
                sidx = jnp.minimum(si_v[pl.ds(v * 16, 16)], nmax) * 3
                didx = jnp.minimum(draw, nmax) * 3
                acc = jnp.zeros((16,), jnp.float32)
                for dim in range(3):
                    off = jnp.full((16,), dim, jnp.int32)
                    xs = plsc.load_gather(tab_v, [sidx + off])
                    xd = plsc.load_gather(tab_v, [didx + off])
                    dd = xd - xs
                    acc = acc + dd * dd
                acc = jnp.where(draw >= n, 1e12, acc)
                o_v[pl.ds(v * 16, 16)] = acc
            pltpu.sync_copy(o_v, out_hbm.at[pl.ds(e0, ch)])
            return carry

        lax.fori_loop(0, n_chunks, chunk, 0)

    return k(pos_flat, src_p, dst_p)


def _sc_gather(h, src_p):
    n, d = h.shape
    ep = src_p.shape[0]
    per_tile = ep // 32
    ch = 64
    n_chunks = per_tile // ch

    @functools.partial(
        pl.kernel,
        mesh=plsc.VectorSubcoreMesh(**_SC_MESH),
        compiler_params=pltpu.CompilerParams(needs_layout_passes=False),
        out_type=jax.ShapeDtypeStruct((ep, d), jnp.float32),
        scratch_types=[pltpu.VMEM((ch,), jnp.int32),
                       pltpu.VMEM((ch, d), jnp.float32),
                       pltpu.SemaphoreType.DMA],
    )
    def k(h_hbm, src_hbm, out_hbm, idx_v, rows_v, sem):
        wid = lax.axis_index("s") * 2 + lax.axis_index("c")
        base = wid * per_tile

        def chunk(j, carry):
            e0 = base + j * ch
            pltpu.sync_copy(src_hbm.at[pl.ds(e0, ch)], idx_v)
            pltpu.async_copy(h_hbm.at[idx_v], rows_v, sem).wait()
            pltpu.sync_copy(rows_v, out_hbm.at[pl.ds(e0, ch)])
            return carry

        lax.fori_loop(0, n_chunks, chunk, 0)

    return k(h, src_p)


def kernel(pos, cell, atomic_numbers, num_atoms, batch, edge_index, atom_emb,
           W_rbf, W_msg, W_upd, W1, b1, W2, b2, Wd, bd):
    n = pos.shape[0]
    b = cell.shape[0]
    e = edge_index.shape[1]
    d = W_rbf.shape[1]
    nb = W_msg.shape[0]
    apg = n // b

    te = 256
    npad = ((n + te - 1) // te) * te
    nblk = npad // te
    ep = ((e + nblk * te + 2047) // 2048) * 2048
    grid = ep // te

    src = edge_index[0].astype(jnp.int32)
    dst = edge_index[1].astype(jnp.int32)
    perm = jnp.argsort(dst)
    src_s = src[perm]
    dst_s = dst[perm]
    offs = jnp.searchsorted(
        dst_s, jnp.arange(nblk + 1, dtype=jnp.int32) * te).astype(jnp.int32)
    cnt = offs[1:] - offs[:-1]
    ntiles = jnp.maximum(1, (cnt + te - 1) // te)
    tile_start = jnp.concatenate(
        [jnp.zeros((1,), jnp.int32), jnp.cumsum(ntiles).astype(jnp.int32)])
    w_map = (jnp.searchsorted(tile_start[:nblk], jnp.arange(grid),
                              side='right') - 1).astype(jnp.int32)
    blk_of_e = dst_s // te
    i_in_blk = jnp.arange(e, dtype=jnp.int32) - offs[blk_of_e]
    p_of_e = (tile_start[blk_of_e] + i_in_blk // te) * te + i_in_blk % te
    src_p = jnp.zeros((ep,), jnp.int32).at[p_of_e].set(src_s)
    dst_p = jnp.full((ep,), n, jnp.int32).at[p_of_e].set(dst_s)
    dstloc = jnp.zeros((ep,), jnp.int32).at[p_of_e].set(dst_s - blk_of_e * te)

    cell_rep = jnp.repeat(cell, apg, axis=0)
    emb_pad = jnp.zeros((128, d), jnp.float32).at[:atom_emb.shape[0]].set(
        atom_emb)

    pos_cart = _tc_poscart(pos, cell_rep)
    s2 = _sc_dist(pos_cart.reshape(n * 3), n, src_p, dst_p)
    s2col = s2.reshape(ep, 1)
    dlcol = dstloc.reshape(ep, 1)
    h = _tc_embed(atomic_numbers, emb_pad)

    for blk in range(nb):
        hsrc = _sc_gather(h, src_p)
        agg = _tc_msg_agg(w_map, s2col, hsrc, dlcol, W_rbf, W_msg[blk],
                          npad, te)
        h = _tc_update(h, agg, W_upd[blk])

    pooled = _tc_pool(h, b)
    pred, repr_out = _tc_head(pooled, W1, b1, W2, b2, Wd, bd)
    return pred, repr_out
